# baseline (speedup 1.0000x reference)
import functools

import jax
import jax.numpy as jnp
from jax import lax
from jax.experimental import pallas as pl
from jax.experimental.pallas import tpu as pltpu
from jax.experimental.pallas import tpu_sc as plsc

N = 10000
NP = 10240
E = 320000
NC, NS = 2, 16
NW = NC * NS
L = 16
EPT = 10240
EP = EPT * NW
GC = 128
NCHT = EPT // GC
RPT = NP // NS
BR = 1024
GRID = NP // BR


@functools.lru_cache(maxsize=None)
def _make_agg(ROWW: int):
    mesh = plsc.VectorSubcoreMesh(
        core_axis_name="c", subcore_axis_name="s", num_cores=NC, num_subcores=NS
    )
    ZR = 64

    NSLOT = 4

    @functools.partial(
        pl.kernel,
        out_type=jax.ShapeDtypeStruct((NC, NP, ROWW), jnp.float32),
        mesh=mesh,
        scratch_types=(
            [pltpu.VMEM((GC,), jnp.int32) for _ in range(NSLOT)]
            + [pltpu.VMEM((GC,), jnp.int32) for _ in range(NSLOT)]
            + [pltpu.VMEM((GC,), jnp.float32) for _ in range(NSLOT)]
            + [pltpu.VMEM((GC, ROWW), jnp.float32) for _ in range(2)]
            + [pltpu.VMEM((ZR, ROWW), jnp.float32)]
            + [pltpu.VMEM_SHARED((NP, ROWW), jnp.float32)]
            + [pltpu.SemaphoreType.DMA for _ in range(2 + NSLOT + 2)]
        ),
        compiler_params=pltpu.CompilerParams(
            needs_layout_passes=False, use_tc_tiling_on_sc=False
        ),
    )
    def agg(p_hbm, src_hbm, dst_hbm, w_hbm, out_hbm, *sc):
        srcb = sc[0:NSLOT]
        dstb = sc[NSLOT:2 * NSLOT]
        wb = sc[2 * NSLOT:3 * NSLOT]
        rows = sc[3 * NSLOT:3 * NSLOT + 2]
        zbuf = sc[3 * NSLOT + 2]
        acc = sc[3 * NSLOT + 3]
        gsem = sc[3 * NSLOT + 4:3 * NSLOT + 6]
        isem = sc[3 * NSLOT + 6:3 * NSLOT + 6 + NSLOT]
        ssem = sc[3 * NSLOT + 6 + NSLOT:]

        cid = lax.axis_index("c")
        sid = lax.axis_index("s")
        wid = sid * NC + cid
        ebase = wid * EPT

        def idx_fire(c, r):
            e0 = ebase + c * GC
            pltpu.async_copy(src_hbm.at[pl.ds(e0, GC)], srcb[r], isem[r])
            pltpu.async_copy(dst_hbm.at[pl.ds(e0, GC)], dstb[r], isem[r])
            pltpu.async_copy(w_hbm.at[pl.ds(e0, GC)], wb[r], isem[r])

        def idx_wait(r):
            pltpu.make_async_copy(src_hbm.at[pl.ds(0, GC)], srcb[r], isem[r]).wait()
            pltpu.make_async_copy(dst_hbm.at[pl.ds(0, GC)], dstb[r], isem[r]).wait()
            pltpu.make_async_copy(w_hbm.at[pl.ds(0, GC)], wb[r], isem[r]).wait()

        def g_fire(b, r):
            pltpu.async_copy(p_hbm.at[srcb[r]], rows[b], gsem[b])

        def g_wait(b):
            pltpu.make_async_copy(
                p_hbm.at[pl.ds(0, GC)], rows[b], gsem[b]).wait()

        def s_fire(b, r):
            pltpu.async_copy(rows[b], acc.at[dstb[r]], ssem[b], add=True)

        def s_wait(b):
            pltpu.make_async_copy(
                p_hbm.at[pl.ds(0, GC)], rows[b], ssem[b]).wait()

        zero = jnp.zeros((L,), jnp.float32)

        @plsc.parallel_loop(0, ZR * ROWW // L, unroll=8)
        def _(i):
            r = i // (ROWW // L)
            j = lax.rem(i, ROWW // L)
            zbuf[r, pl.ds(j * L, L)] = zero

        for z in range(RPT // ZR):
            pltpu.sync_copy(zbuf, acc.at[pl.ds(sid * RPT + z * ZR, ZR)])
        plsc.subcore_barrier()

        idx_fire(0, 0)
        idx_fire(1, 1)
        idx_wait(0)
        g_fire(0, 0)

        def quad_body(t, _):
            for k in range(NSLOT):
                c = t * NSLOT + k
                b = k % 2
                r = k
                rn = (k + 1) % NSLOT
                g_wait(b)

                @pl.when(c >= 1)
                def _():
                    s_wait(1 - b)

                @pl.when(c + 1 < NCHT)
                def _():
                    idx_wait(rn)
                    g_fire(1 - b, rn)

                @plsc.parallel_loop(0, GC // L, unroll=1)
                def _(g):
                    wv = wb[r][pl.ds(g * L, L)]
                    for e in range(L):
                        ws = jnp.broadcast_to(wv[e], (L,))
                        i = g * L + e
                        for m in range(ROWW // L):
                            rows[b][i, pl.ds(m * L, L)] = (
                                rows[b][i, pl.ds(m * L, L)] * ws
                            )

                s_fire(b, r)

                @pl.when(c + 2 < NCHT)
                def _():
                    idx_fire(c + 2, (k + 2) % NSLOT)

            return 0

        lax.fori_loop(0, NCHT // NSLOT, quad_body, 0)
        s_wait((NCHT - 1) % 2)

        plsc.subcore_barrier()
        pltpu.sync_copy(
            acc.at[pl.ds(sid * RPT, RPT)],
            out_hbm.at[cid, pl.ds(sid * RPT, RPT)],
        )

    return agg


def _agg(p, src, dst, w):
    return _make_agg(p.shape[1])(p, src, dst, w)


EPS = EP // NS
WCH = 2048


@functools.lru_cache(maxsize=None)
def _make_wn():
    mesh = plsc.VectorSubcoreMesh(
        core_axis_name="c", subcore_axis_name="s", num_cores=NC, num_subcores=NS
    )
    SLC = NP // NS

    @functools.partial(
        pl.kernel,
        out_type=jax.ShapeDtypeStruct((EP,), jnp.float32),
        mesh=mesh,
        scratch_types=[
            pltpu.VMEM((WCH,), jnp.int32),
            pltpu.VMEM((WCH,), jnp.float32),
            pltpu.VMEM((WCH,), jnp.float32),
            pltpu.VMEM((NP,), jnp.float32),
            pltpu.VMEM((SLC,), jnp.float32),
            pltpu.VMEM((SLC,), jnp.float32),
            pltpu.VMEM_SHARED((NS, NP), jnp.float32),
            pltpu.VMEM_SHARED((NP,), jnp.float32),
        ],
        compiler_params=pltpu.CompilerParams(
            needs_layout_passes=False, use_tc_tiling_on_sc=False
        ),
    )
    def wn_kernel(dst_hbm, w_hbm, wn_hbm,
                  dst_v, w_v, wn_v, wsum_v, accs, tmps, part_sp, wsum_sp):
        cid = lax.axis_index("c")
        sid = lax.axis_index("s")
        wid = sid * NC + cid
        zero = jnp.zeros((L,), jnp.float32)

        @plsc.parallel_loop(0, NP // L, unroll=8)
        def _(i):
            wsum_v[pl.ds(i * L, L)] = zero

        def p1_chunk(k, _):
            e0 = sid * EPS + k * WCH
            pltpu.sync_copy(dst_hbm.at[pl.ds(e0, WCH)], dst_v)
            pltpu.sync_copy(w_hbm.at[pl.ds(e0, WCH)], w_v)

            @plsc.parallel_loop(0, WCH // L, unroll=4)
            def _(g):
                d = dst_v[pl.ds(g * L, L)]
                wv = w_v[pl.ds(g * L, L)]
                plsc.addupdate_scatter(wsum_v, [d], wv)

            return 0

        lax.fori_loop(0, EPS // WCH, p1_chunk, 0)

        pltpu.sync_copy(wsum_v, part_sp.at[sid])
        plsc.subcore_barrier()

        @plsc.parallel_loop(0, SLC // L, unroll=8)
        def _(i):
            accs[pl.ds(i * L, L)] = zero

        for j in range(NS):
            pltpu.sync_copy(part_sp.at[j, pl.ds(sid * SLC, SLC)], tmps)

            @plsc.parallel_loop(0, SLC // L, unroll=8)
            def _(i):
                accs[pl.ds(i * L, L)] = (
                    accs[pl.ds(i * L, L)] + tmps[pl.ds(i * L, L)]
                )

        pltpu.sync_copy(accs, wsum_sp.at[pl.ds(sid * SLC, SLC)])
        plsc.subcore_barrier()
        pltpu.sync_copy(wsum_sp, wsum_v)

        def p3_chunk(k, _):
            e0 = wid * EPT + k * WCH
            pltpu.sync_copy(dst_hbm.at[pl.ds(e0, WCH)], dst_v)
            pltpu.sync_copy(w_hbm.at[pl.ds(e0, WCH)], w_v)

            @plsc.parallel_loop(0, WCH // L, unroll=4)
            def _(g):
                d = dst_v[pl.ds(g * L, L)]
                wv = w_v[pl.ds(g * L, L)]
                ws = plsc.load_gather(wsum_v, [d])
                wn_v[pl.ds(g * L, L)] = wv / jnp.maximum(ws, 1e-6)

            pltpu.sync_copy(wn_v, wn_hbm.at[pl.ds(e0, WCH)])
            return 0

        lax.fori_loop(0, EPT // WCH, p3_chunk, 0)

    return wn_kernel


def _dot(h, W):
    return lax.dot_general(
        h, W, (((1,), (0,)), ((), ())), preferred_element_type=jnp.float32
    )


def _norm(t):
    nrm = jnp.sqrt(jnp.sum(t * t, axis=1, keepdims=True))
    return t / jnp.maximum(nrm, 1e-12)


def _rspec(d):
    return pl.BlockSpec((BR, d), lambda i: (i, 0))


def _pspec(d):
    return pl.BlockSpec((NC, BR, d), lambda i: (0, i, 0))


def _full(shape):
    return pl.BlockSpec(shape, lambda i: tuple(0 for _ in shape))


def _out(d):
    return jax.ShapeDtypeStruct((NP, d), jnp.float32)


def _tc_first(x, Ws, b2, Wn):
    din, dout = Ws.shape

    def body(x_ref, ws_ref, b_ref, wn_ref, s_ref, p_ref):
        h = x_ref[...]
        s_ref[...] = _dot(h, ws_ref[...]) + b_ref[:1]
        p_ref[...] = _dot(h, wn_ref[...])

    return pl.pallas_call(
        body,
        grid=(GRID,),
        in_specs=[_rspec(din), _full((din, dout)), _full((8, dout)),
                  _full((din, dout))],
        out_specs=[_rspec(dout), _rspec(dout)],
        out_shape=[_out(dout), _out(dout)],
    )(x, Ws, b2, Wn)


def _tc_comb(s0, part, res, Ws, Wn, b2, relu, emit_h):
    din, dout = Ws.shape
    have_res = res is not None

    def body(*refs):
        if have_res:
            s_ref, p_ref, x_ref, ws_ref, wn_ref, b_ref, *outs = refs
            rv = x_ref[...]
        else:
            s_ref, p_ref, ws_ref, wn_ref, b_ref, *outs = refs
            rv = None
        t = s_ref[...] + p_ref[0] + p_ref[1]
        if relu:
            t = jnp.maximum(t, 0.0)
        if rv is not None:
            t = t + rv
        h = _norm(t)
        outs[0][...] = _dot(h, ws_ref[...]) + b_ref[:1]
        outs[1][...] = _dot(h, wn_ref[...])
        if emit_h:
            outs[2][...] = h

    in_specs = [_rspec(din), _pspec(din)]
    args = [s0, part]
    if have_res:
        in_specs.append(_rspec(din))
        args.append(res)
    in_specs += [_full((din, dout)), _full((din, dout)), _full((8, dout))]
    args += [Ws, Wn, b2]
    out_specs = [_rspec(dout), _rspec(dout)] + ([_rspec(din)] if emit_h else [])
    out_shape = [_out(dout), _out(dout)] + ([_out(din)] if emit_h else [])
    return pl.pallas_call(
        body, grid=(GRID,), in_specs=in_specs, out_specs=out_specs,
        out_shape=out_shape,
    )(*args)


def _tc_comb3(s2, part, h2):
    d = s2.shape[1]

    def body(s_ref, p_ref, h_ref, o_ref):
        o_ref[...] = _norm(s_ref[...] + p_ref[0] + p_ref[1] + h_ref[...])

    return pl.pallas_call(
        body,
        grid=(GRID,),
        in_specs=[_rspec(d), _pspec(d), _rspec(d)],
        out_specs=_rspec(d),
        out_shape=_out(d),
    )(s2, part, h2)


def kernel(x, edge_index, edge_weights, Ws0, Wn0, b0, Ws1, Wn1, b1, Ws2, Wn2, b2):
    f32 = jnp.float32
    xp = jnp.pad(x, ((0, NP - N), (0, 0)))
    src = jnp.pad(edge_index[0].astype(jnp.int32), (0, EP - E))
    dst = jnp.pad(edge_index[1].astype(jnp.int32), (0, EP - E))
    w = jnp.pad(edge_weights.astype(f32), (0, EP - E))

    b0r = jnp.tile(b0[None, :], (8, 1))
    b1r = jnp.tile(b1[None, :], (8, 1))
    b2r = jnp.tile(b2[None, :], (8, 1))

    wn = _make_wn()(dst, w)
    s0, p0 = _tc_first(xp, Ws0, b0r, Wn0)
    part0 = _agg(p0, src, dst, wn)
    s1, p1 = _tc_comb(s0, part0, xp, Ws1, Wn1, b1r, relu=True, emit_h=False)
    part1 = _agg(p1, src, dst, wn)
    s2, p2, h2 = _tc_comb(s1, part1, None, Ws2, Wn2, b2r, relu=True,
                          emit_h=True)
    part2 = _agg(p2, src, dst, wn)
    h3 = _tc_comb3(s2, part2, h2)
    return h3[:N, :]


# speedup vs baseline: 1.6064x; 1.6064x over previous
import functools

import jax
import jax.numpy as jnp
from jax import lax
from jax.experimental import pallas as pl
from jax.experimental.pallas import tpu as pltpu
from jax.experimental.pallas import tpu_sc as plsc

N = 10000
NP = 10240
E = 320000
NC, NS = 2, 16
NW = NC * NS
L = 16
CE = 16000
BN = 2048
NBLK = NP // BN


@functools.lru_cache(maxsize=None)
def _make_agg(D: int, with_wsum: bool):
    dpw = D // NW
    nchunks = E // CE
    ngroups = CE // L
    mesh = plsc.VectorSubcoreMesh(
        core_axis_name="c", subcore_axis_name="s", num_cores=NC, num_subcores=NS
    )
    out_type = [jax.ShapeDtypeStruct((D * NP,), jnp.float32)]
    scratch = [
        pltpu.VMEM((dpw * NP,), jnp.float32),
        pltpu.VMEM((dpw * NP,), jnp.float32),
        pltpu.VMEM((CE,), jnp.int32),
        pltpu.VMEM((CE,), jnp.float32),
    ]
    if with_wsum:
        out_type.append(jax.ShapeDtypeStruct((NP,), jnp.float32))
        scratch.append(pltpu.VMEM((NP,), jnp.float32))

    @functools.partial(
        pl.kernel, out_type=tuple(out_type), mesh=mesh, scratch_types=scratch,
        compiler_params=pltpu.CompilerParams(needs_layout_passes=False),
    )
    def agg(*refs):
        if with_wsum:
            (p_hbm, sd_hbm, w_hbm, out_hbm, wsum_hbm,
             pcols, ocols, sd_v, w_v, wacc) = refs
        else:
            (p_hbm, sd_hbm, w_hbm, out_hbm,
             pcols, ocols, sd_v, w_v) = refs
            wacc = None
        wid = lax.axis_index("s") * NC + lax.axis_index("c")
        base = wid * dpw * NP
        pltpu.sync_copy(p_hbm.at[pl.ds(base, dpw * NP)], pcols)

        zero = jnp.zeros((L,), jnp.float32)

        @plsc.parallel_loop(0, dpw * NP // L, unroll=8)
        def _(i):
            ocols[pl.ds(i * L, L)] = zero

        if with_wsum:
            @plsc.parallel_loop(0, NP // L, unroll=8)
            def _(i):
                wacc[pl.ds(i * L, L)] = zero

        def chunk_body(k, _):
            e0 = k * CE
            pltpu.sync_copy(sd_hbm.at[pl.ds(e0, CE)], sd_v)
            pltpu.sync_copy(w_hbm.at[pl.ds(e0, CE)], w_v)

            @plsc.parallel_loop(0, ngroups, unroll=4)
            def _(g):
                off = g * L
                sd = sd_v[pl.ds(off, L)]
                s = lax.bitwise_and(sd, jnp.int32(0x3FFF))
                d = lax.shift_right_logical(sd, jnp.int32(14))
                wv = w_v[pl.ds(off, L)]
                for c in range(dpw):
                    vals = plsc.load_gather(pcols, [s + (c * NP)])
                    plsc.addupdate_scatter(ocols, [d + (c * NP)], vals * wv)
                if with_wsum:
                    plsc.addupdate_scatter(wacc, [d], wv)

            return 0

        lax.fori_loop(0, nchunks, chunk_body, 0)

        pltpu.sync_copy(ocols, out_hbm.at[pl.ds(base, dpw * NP)])
        if with_wsum:
            @pl.when(wid == 0)
            def _():
                pltpu.sync_copy(wacc, wsum_hbm)

    return agg


def _agg_wsum(p, sd, w):
    D = p.shape[0]
    out, wsum = _make_agg(D, True)(p.reshape(-1), sd, w)
    return out.reshape(D, NP), wsum


def _agg(p, sd, w):
    D = p.shape[0]
    (out,) = _make_agg(D, False)(p.reshape(-1), sd, w)
    return out.reshape(D, NP)


def _proj(W, h):
    return lax.dot_general(
        W, h, (((0,), (0,)), ((), ())), preferred_element_type=jnp.float32
    )


def _combine(s, agg, winv, res, relu):
    t = s + agg * winv
    if relu:
        t = jnp.maximum(t, 0.0)
    if res is not None:
        t = t + res
    nrm = jnp.sqrt(jnp.sum(t * t, axis=0, keepdims=True))
    return t / jnp.maximum(nrm, 1e-12)


def _winv(wsum_blk):
    return 1.0 / jnp.maximum(wsum_blk[0], 1e-6)


def _bspec(d):
    return pl.BlockSpec((d, BN), lambda i: (0, i))


_WSPEC = pl.BlockSpec((1, 1, BN), lambda i: (i, 0, 0))


def _full(shape):
    return pl.BlockSpec(shape, lambda i: tuple(0 for _ in shape))


def _tc_first(xT, Ws, Wn, b2):
    din, dout = Ws.shape

    def body(x_ref, ws_ref, wn_ref, b_ref, s_ref, p_ref):
        h = x_ref[...]
        s_ref[...] = _proj(ws_ref[...], h) + b_ref[:, :1]
        p_ref[...] = _proj(wn_ref[...], h)

    return pl.pallas_call(
        body,
        grid=(NBLK,),
        in_specs=[_bspec(din), _full((din, dout)), _full((din, dout)),
                  _full((dout, 128))],
        out_specs=[_bspec(dout), _bspec(dout)],
        out_shape=[jax.ShapeDtypeStruct((dout, NP), jnp.float32)] * 2,
    )(xT, Ws, Wn, b2)


def _tc_mid(s, agg, wsum3, res, Ws, Wn, b2, relu, emit_h):
    din, dout = Ws.shape
    have_res = res is not None

    def body(*refs):
        if have_res:
            s_ref, a_ref, w_ref, r_ref, ws_ref, wn_ref, b_ref, *outs = refs
            rv = r_ref[...]
        else:
            s_ref, a_ref, w_ref, ws_ref, wn_ref, b_ref, *outs = refs
            rv = None
        h = _combine(s_ref[...], a_ref[...], _winv(w_ref[...]), rv, relu)
        outs[0][...] = _proj(ws_ref[...], h) + b_ref[:, :1]
        outs[1][...] = _proj(wn_ref[...], h)
        if emit_h:
            outs[2][...] = h

    in_specs = [_bspec(din), _bspec(din), _WSPEC]
    args = [s, agg, wsum3]
    if have_res:
        in_specs.append(_bspec(din))
        args.append(res)
    in_specs += [_full((din, dout)), _full((din, dout)), _full((dout, 128))]
    args += [Ws, Wn, b2]
    n_out = 3 if emit_h else 2
    out_specs = [_bspec(dout), _bspec(dout)] + ([_bspec(din)] if emit_h else [])
    out_shape = ([jax.ShapeDtypeStruct((dout, NP), jnp.float32)] * 2
                 + ([jax.ShapeDtypeStruct((din, NP), jnp.float32)] if emit_h else []))
    return pl.pallas_call(
        body, grid=(NBLK,), in_specs=in_specs, out_specs=out_specs,
        out_shape=out_shape,
    )(*args)


def _tc_last(s, agg, wsum3, res):
    d = s.shape[0]

    def body(s_ref, a_ref, w_ref, r_ref, o_ref):
        o_ref[...] = _combine(
            s_ref[...], a_ref[...], _winv(w_ref[...]), r_ref[...], relu=False
        )

    return pl.pallas_call(
        body,
        grid=(NBLK,),
        in_specs=[_bspec(d), _bspec(d), _WSPEC, _bspec(d)],
        out_specs=_bspec(d),
        out_shape=jax.ShapeDtypeStruct((d, NP), jnp.float32),
    )(s, agg, wsum3, res)


def kernel(x, edge_index, edge_weights, Ws0, Wn0, b0, Ws1, Wn1, b1, Ws2, Wn2, b2):
    xT = jnp.pad(x.T, ((0, 0), (0, NP - N)))
    src = edge_index[0].astype(jnp.int32)
    dst = edge_index[1].astype(jnp.int32)
    sd = src | (dst << 14)
    w = edge_weights.astype(jnp.float32)
    b0r = jnp.tile(b0[:, None], (1, 128))
    b1r = jnp.tile(b1[:, None], (1, 128))
    b2r = jnp.tile(b2[:, None], (1, 128))

    s0, p0 = _tc_first(xT, Ws0, Wn0, b0r)
    agg0, wsum = _agg_wsum(p0, sd, w)
    wsum3 = wsum.reshape(NBLK, 1, BN)
    s1, p1 = _tc_mid(s0, agg0, wsum3, xT, Ws1, Wn1, b1r, relu=True, emit_h=False)
    agg1 = _agg(p1, sd, w)
    s2, p2, h2 = _tc_mid(s1, agg1, wsum3, None, Ws2, Wn2, b2r, relu=True,
                         emit_h=True)
    agg2 = _agg(p2, sd, w)
    h3 = _tc_last(s2, agg2, wsum3, h2)
    return h3[:, :N].T


# speedup vs baseline: 1.6431x; 1.0229x over previous
import functools

import jax
import jax.numpy as jnp
from jax import lax
from jax.experimental import pallas as pl
from jax.experimental.pallas import tpu as pltpu
from jax.experimental.pallas import tpu_sc as plsc

N = 10000
NP = 10240
E = 320000
NC, NS = 2, 16
NW = NC * NS
L = 16
CE = 16000
BN = 2048
NBLK = NP // BN


@functools.lru_cache(maxsize=None)
def _make_agg(D: int, with_wsum: bool):
    dpw = D // NW
    CE = 16000 if D == 128 else 32000
    nchunks = E // CE
    ngroups = CE // L
    mesh = plsc.VectorSubcoreMesh(
        core_axis_name="c", subcore_axis_name="s", num_cores=NC, num_subcores=NS
    )
    out_type = [jax.ShapeDtypeStruct((D * NP,), jnp.float32)]
    scratch = [
        pltpu.VMEM((dpw * NP,), jnp.float32),
        pltpu.VMEM((dpw * NP,), jnp.float32),
        pltpu.VMEM((CE,), jnp.int32),
        pltpu.VMEM((CE,), jnp.float32),
    ]
    if with_wsum:
        out_type.append(jax.ShapeDtypeStruct((NP,), jnp.float32))
        scratch.append(pltpu.VMEM((NP,), jnp.float32))

    @functools.partial(
        pl.kernel, out_type=tuple(out_type), mesh=mesh, scratch_types=scratch,
        compiler_params=pltpu.CompilerParams(needs_layout_passes=False),
    )
    def agg(*refs):
        if with_wsum:
            (p_hbm, sd_hbm, w_hbm, out_hbm, wsum_hbm,
             pcols, ocols, sd_v, w_v, wacc) = refs
        else:
            (p_hbm, sd_hbm, w_hbm, out_hbm,
             pcols, ocols, sd_v, w_v) = refs
            wacc = None
        wid = lax.axis_index("s") * NC + lax.axis_index("c")
        base = wid * dpw * NP
        pltpu.sync_copy(p_hbm.at[pl.ds(base, dpw * NP)], pcols)

        zero = jnp.zeros((L,), jnp.float32)

        @plsc.parallel_loop(0, dpw * NP // L, unroll=8)
        def _(i):
            ocols[pl.ds(i * L, L)] = zero

        if with_wsum:
            @plsc.parallel_loop(0, NP // L, unroll=8)
            def _(i):
                wacc[pl.ds(i * L, L)] = zero

        def chunk_body(k, _):
            e0 = k * CE
            pltpu.sync_copy(sd_hbm.at[pl.ds(e0, CE)], sd_v)
            pltpu.sync_copy(w_hbm.at[pl.ds(e0, CE)], w_v)

            @plsc.parallel_loop(0, ngroups, unroll=4)
            def _(g):
                off = g * L
                sd = sd_v[pl.ds(off, L)]
                s = lax.bitwise_and(sd, jnp.int32(0x3FFF))
                d = lax.shift_right_logical(sd, jnp.int32(14))
                wv = w_v[pl.ds(off, L)]
                for c in range(dpw):
                    vals = plsc.load_gather(pcols, [s + (c * NP)])
                    plsc.addupdate_scatter(ocols, [d + (c * NP)], vals * wv)
                if with_wsum:
                    plsc.addupdate_scatter(wacc, [d], wv)

            return 0

        lax.fori_loop(0, nchunks, chunk_body, 0)

        pltpu.sync_copy(ocols, out_hbm.at[pl.ds(base, dpw * NP)])
        if with_wsum:
            @pl.when(wid == 0)
            def _():
                pltpu.sync_copy(wacc, wsum_hbm)

    return agg


def _agg_wsum(p, sd, w):
    D = p.shape[0]
    out, wsum = _make_agg(D, True)(p.reshape(-1), sd, w)
    return out.reshape(D, NP), wsum


def _agg(p, sd, w):
    D = p.shape[0]
    (out,) = _make_agg(D, False)(p.reshape(-1), sd, w)
    return out.reshape(D, NP)


def _proj(W, h):
    return lax.dot_general(
        W, h, (((0,), (0,)), ((), ())), preferred_element_type=jnp.float32
    )


def _combine(s, agg, winv, res, relu):
    t = s + agg * winv
    if relu:
        t = jnp.maximum(t, 0.0)
    if res is not None:
        t = t + res
    nrm = jnp.sqrt(jnp.sum(t * t, axis=0, keepdims=True))
    return t / jnp.maximum(nrm, 1e-12)


def _winv(wsum_blk):
    return 1.0 / jnp.maximum(wsum_blk[0], 1e-6)


def _bspec(d):
    return pl.BlockSpec((d, BN), lambda i: (0, i))


_WSPEC = pl.BlockSpec((1, 1, BN), lambda i: (i, 0, 0))


def _full(shape):
    return pl.BlockSpec(shape, lambda i: tuple(0 for _ in shape))


def _tc_first(xT, Ws, Wn, b2):
    din, dout = Ws.shape

    def body(x_ref, ws_ref, wn_ref, b_ref, s_ref, p_ref):
        h = x_ref[...]
        s_ref[...] = _proj(ws_ref[...], h) + b_ref[:, :1]
        p_ref[...] = _proj(wn_ref[...], h)

    return pl.pallas_call(
        body,
        grid=(NBLK,),
        in_specs=[_bspec(din), _full((din, dout)), _full((din, dout)),
                  _full((dout, 128))],
        out_specs=[_bspec(dout), _bspec(dout)],
        out_shape=[jax.ShapeDtypeStruct((dout, NP), jnp.float32)] * 2,
    )(xT, Ws, Wn, b2)


def _tc_mid(s, agg, wsum3, res, Ws, Wn, b2, relu, emit_h):
    din, dout = Ws.shape
    have_res = res is not None

    def body(*refs):
        if have_res:
            s_ref, a_ref, w_ref, r_ref, ws_ref, wn_ref, b_ref, *outs = refs
            rv = r_ref[...]
        else:
            s_ref, a_ref, w_ref, ws_ref, wn_ref, b_ref, *outs = refs
            rv = None
        h = _combine(s_ref[...], a_ref[...], _winv(w_ref[...]), rv, relu)
        outs[0][...] = _proj(ws_ref[...], h) + b_ref[:, :1]
        outs[1][...] = _proj(wn_ref[...], h)
        if emit_h:
            outs[2][...] = h

    in_specs = [_bspec(din), _bspec(din), _WSPEC]
    args = [s, agg, wsum3]
    if have_res:
        in_specs.append(_bspec(din))
        args.append(res)
    in_specs += [_full((din, dout)), _full((din, dout)), _full((dout, 128))]
    args += [Ws, Wn, b2]
    n_out = 3 if emit_h else 2
    out_specs = [_bspec(dout), _bspec(dout)] + ([_bspec(din)] if emit_h else [])
    out_shape = ([jax.ShapeDtypeStruct((dout, NP), jnp.float32)] * 2
                 + ([jax.ShapeDtypeStruct((din, NP), jnp.float32)] if emit_h else []))
    return pl.pallas_call(
        body, grid=(NBLK,), in_specs=in_specs, out_specs=out_specs,
        out_shape=out_shape,
    )(*args)


def _tc_last(s, agg, wsum3, res):
    d = s.shape[0]

    def body(s_ref, a_ref, w_ref, r_ref, o_ref):
        o_ref[...] = _combine(
            s_ref[...], a_ref[...], _winv(w_ref[...]), r_ref[...], relu=False
        )

    return pl.pallas_call(
        body,
        grid=(NBLK,),
        in_specs=[_bspec(d), _bspec(d), _WSPEC, _bspec(d)],
        out_specs=_bspec(d),
        out_shape=jax.ShapeDtypeStruct((d, NP), jnp.float32),
    )(s, agg, wsum3, res)


def kernel(x, edge_index, edge_weights, Ws0, Wn0, b0, Ws1, Wn1, b1, Ws2, Wn2, b2):
    xT = jnp.pad(x.T, ((0, 0), (0, NP - N)))
    src = edge_index[0].astype(jnp.int32)
    dst = edge_index[1].astype(jnp.int32)
    sd = src | (dst << 14)
    w = edge_weights.astype(jnp.float32)
    b0r = jnp.tile(b0[:, None], (1, 128))
    b1r = jnp.tile(b1[:, None], (1, 128))
    b2r = jnp.tile(b2[:, None], (1, 128))

    s0, p0 = _tc_first(xT, Ws0, Wn0, b0r)
    agg0, wsum = _agg_wsum(p0, sd, w)
    wsum3 = wsum.reshape(NBLK, 1, BN)
    s1, p1 = _tc_mid(s0, agg0, wsum3, xT, Ws1, Wn1, b1r, relu=True, emit_h=False)
    agg1 = _agg(p1, sd, w)
    s2, p2, h2 = _tc_mid(s1, agg1, wsum3, None, Ws2, Wn2, b2r, relu=True,
                         emit_h=True)
    agg2 = _agg(p2, sd, w)
    h3 = _tc_last(s2, agg2, wsum3, h2)
    return h3[:, :N].T
